# Initial kernel scaffold; baseline (speedup 1.0000x reference)
import jax
import jax.numpy as jnp
from jax.experimental import pallas as pl


def kernel(edge_index, edge_weight, user_emb, item_emb):
    raise NotImplementedError("write your pallas kernel here")


# speedup vs baseline: 2.3736x; 2.3736x over previous
import functools

import jax
import jax.numpy as jnp
from jax import lax
from jax.experimental import pallas as pl
from jax.experimental.pallas import tpu as pltpu
from jax.experimental.pallas import tpu_sc as plsc

N_U = 25000
N_I = 25000
EMB = 64
N_LAYERS = 3
E = 800000

HALF = 25088
NTOT = 2 * HALF
PADG = HALF - N_U
B = 128
NB_TILE = 391
E_TILE = NB_TILE * B
E_PAD = 16 * E_TILE
STRIPE = HALF // 16
CH = 56
NCH = STRIPE // CH
TRASH = HALF
ACC_ROWS = HALF + 8


def _layer_body(scale_in, packed_hbm, emb_hbm, sum_hbm, embo_hbm, sumo_hbm,
                acc, pkbuf, rows, dstloc, buf, buf2, sem):
    c = lax.axis_index("c")
    s = lax.axis_index("s")

    z = jnp.zeros((16,), jnp.float32)

    def zrow(r, carry):
        for q in range(EMB // 16):
            buf[r, pl.ds(q * 16, 16)] = z
        return carry

    lax.fori_loop(0, CH, zrow, 0)
    for k in range(NCH):
        pltpu.sync_copy(buf, acc.at[pl.ds(s * STRIPE + k * CH, CH)])
    plsc.subcore_barrier()

    base_b = s * NB_TILE

    def batch(b, carry):
        pltpu.sync_copy(packed_hbm.at[base_b + b], pkbuf)
        pltpu.async_copy(emb_hbm.at[pkbuf.at[0]], rows, sem).wait()

        def group(g, gc):
            dv = pkbuf[1, pl.ds(g * 16, 16)]
            dl = dv - c * HALF
            ok = (dl >= 0) & (dl < HALF)
            dstloc[0, pl.ds(g * 16, 16)] = jnp.where(ok, dl, TRASH)
            wv = lax.bitcast_convert_type(pkbuf[2, pl.ds(g * 16, 16)],
                                          jnp.float32)
            for j in range(16):
                ws = wv.at[jnp.full((16,), j, jnp.int32)].get(
                    mode="promise_in_bounds")
                e = g * 16 + j
                for q in range(EMB // 16):
                    sl = pl.ds(q * 16, 16)
                    rows[e, sl] = rows[e, sl] * ws
            return gc

        lax.fori_loop(0, B // 16, group, 0)
        pltpu.sync_copy(rows, acc.at[dstloc.at[0]], add=True)
        return carry

    lax.fori_loop(0, NB_TILE, batch, 0)
    plsc.subcore_barrier()

    row0 = c * HALF + s * STRIPE
    pltpu.sync_copy(acc.at[pl.ds(s * STRIPE, STRIPE)],
                    embo_hbm.at[pl.ds(row0, STRIPE)])
    for k in range(NCH):
        pltpu.sync_copy(acc.at[pl.ds(s * STRIPE + k * CH, CH)], buf)
        pltpu.sync_copy(sum_hbm.at[pl.ds(row0 + k * CH, CH)], buf2)

        def srow(r, carry):
            for q in range(EMB // 16):
                sl = pl.ds(q * 16, 16)
                buf2[r, sl] = buf2[r, sl] * scale_in + buf[r, sl] * 0.25
            return carry

        lax.fori_loop(0, CH, srow, 0)
        pltpu.sync_copy(buf2, sumo_hbm.at[pl.ds(row0 + k * CH, CH)])


def _make_layer(scale_in):
    mesh = plsc.VectorSubcoreMesh(core_axis_name="c", subcore_axis_name="s")
    return pl.kernel(
        functools.partial(_layer_body, scale_in),
        out_type=(
            jax.ShapeDtypeStruct((NTOT, EMB), jnp.float32),
            jax.ShapeDtypeStruct((NTOT, EMB), jnp.float32),
        ),
        mesh=mesh,
        compiler_params=pltpu.CompilerParams(use_tc_tiling_on_sc=False),
        scratch_types=[
            pltpu.VMEM_SHARED((ACC_ROWS, EMB), jnp.float32),
            pltpu.VMEM((3, B), jnp.int32),
            pltpu.VMEM((B, EMB), jnp.float32),
            pltpu.VMEM((1, B), jnp.int32),
            pltpu.VMEM((CH, EMB), jnp.float32),
            pltpu.VMEM((CH, EMB), jnp.float32),
            pltpu.SemaphoreType.DMA,
        ],
        name=f"lightgcn_layer_s{scale_in}",
    )


def kernel(edge_index, edge_weight, user_emb, item_emb):
    src = edge_index[1].astype(jnp.int32)
    dst = edge_index[0].astype(jnp.int32)
    src_p = src + PADG * (src >= N_U).astype(jnp.int32)
    dst_p = dst + PADG * (dst >= N_U).astype(jnp.int32)
    wbits = lax.bitcast_convert_type(edge_weight.astype(jnp.float32), jnp.int32)
    packed = jnp.stack([src_p, dst_p, wbits])
    packed = jnp.pad(packed, ((0, 0), (0, E_PAD - E)))
    packed = packed.reshape(3, 16 * NB_TILE, B).transpose(1, 0, 2)

    zpad = jnp.zeros((PADG, EMB), jnp.float32)
    emb0 = jnp.concatenate([user_emb, zpad, item_emb, zpad], axis=0)

    l_first = _make_layer(0.25)
    l_rest = _make_layer(1.0)
    emb1, sum1 = l_first(packed, emb0, emb0)
    emb2, sum2 = l_rest(packed, emb1, sum1)
    _, sum3 = l_rest(packed, emb2, sum2)
    return (sum3[:N_U], sum3[HALF:HALF + N_I])


# speedup vs baseline: 6.9880x; 2.9441x over previous
import functools

import jax
import jax.numpy as jnp
from jax import lax
from jax.experimental import pallas as pl
from jax.experimental.pallas import tpu as pltpu
from jax.experimental.pallas import tpu_sc as plsc

N_U = 25000
N_I = 25000
EMB = 64
HEMB = 32
E = 800000

HALF = 25088
NTOT = 2 * HALF
PADG = HALF - N_U
B = 128
NBT = 392
NBALL = 16 * NBT
E_PAD = NBALL * B
STRIPE = NTOT // 16
ZCH = 196
NZ = STRIPE // ZCH


def _scale_rows(rows, pk):
    def group(g, carry):
        wv = lax.bitcast_convert_type(pk[2, pl.ds(g * 16, 16)], jnp.float32)
        for j in range(16):
            ws = wv.at[jnp.full((16,), j, jnp.int32)].get(
                mode="promise_in_bounds")
            e = g * 16 + j
            for q in range(HEMB // 16):
                sl = pl.ds(q * 16, 16)
                rows[e, sl] = rows[e, sl] * ws
        return carry

    lax.fori_loop(0, B // 16, group, 0)


def _layer_body(final, packed_hbm, tin_hbm, sin_hbm, tout_hbm, sout_hbm,
                acc, pk0, pk1, rows0, rows1, zbuf, sbuf, sem0, sem1):
    c = lax.axis_index("c")
    s = lax.axis_index("s")

    z = jnp.zeros((16,), jnp.float32)

    def zrow(r, carry):
        for q in range(HEMB // 16):
            zbuf[r, pl.ds(q * 16, 16)] = z
        return carry

    lax.fori_loop(0, ZCH, zrow, 0)
    for k in range(NZ):
        pltpu.sync_copy(zbuf, acc.at[pl.ds(s * STRIPE + k * ZCH, ZCH)])
    plsc.subcore_barrier()

    base = c * NBALL + s * NBT

    pltpu.sync_copy(packed_hbm.at[base], pk0)
    g0 = pltpu.async_copy(tin_hbm.at[pk0.at[0]], rows0, sem0)

    def pair(i, carry):
        b = 2 * i
        pltpu.sync_copy(packed_hbm.at[base + b + 1], pk1)
        pltpu.async_copy(tin_hbm.at[pk1.at[0]], rows1, sem1)
        pltpu.make_async_copy(tin_hbm.at[pk0.at[0]], rows0, sem0).wait()
        _scale_rows(rows0, pk0)
        pltpu.sync_copy(rows0, acc.at[pk0.at[1]], add=True)

        @pl.when(i < NBT // 2 - 1)
        def _():
            pltpu.sync_copy(packed_hbm.at[base + b + 2], pk0)
            pltpu.async_copy(tin_hbm.at[pk0.at[0]], rows0, sem0)

        pltpu.make_async_copy(tin_hbm.at[pk1.at[0]], rows1, sem1).wait()
        _scale_rows(rows1, pk1)
        pltpu.sync_copy(rows1, acc.at[pk1.at[1]], add=True)
        return carry

    lax.fori_loop(0, NBT // 2, pair, 0)
    plsc.subcore_barrier()

    row0 = c * NTOT + s * STRIPE
    if not final:
        pltpu.sync_copy(acc.at[pl.ds(s * STRIPE, STRIPE)],
                        tout_hbm.at[pl.ds(row0, STRIPE)])
    for k in range(NZ):
        pltpu.sync_copy(acc.at[pl.ds(s * STRIPE + k * ZCH, ZCH)], zbuf)
        pltpu.sync_copy(sin_hbm.at[pl.ds(row0 + k * ZCH, ZCH)], sbuf)

        def srow(r, carry):
            for q in range(HEMB // 16):
                sl = pl.ds(q * 16, 16)
                v = sbuf[r, sl] + zbuf[r, sl]
                if final:
                    v = v * 0.25
                sbuf[r, sl] = v
            return carry

        lax.fori_loop(0, ZCH, srow, 0)
        pltpu.sync_copy(sbuf, sout_hbm.at[pl.ds(row0 + k * ZCH, ZCH)])


def _make_layer(final):
    mesh = plsc.VectorSubcoreMesh(core_axis_name="c", subcore_axis_name="s")
    return pl.kernel(
        functools.partial(_layer_body, final),
        out_type=(
            jax.ShapeDtypeStruct((2 * NTOT, HEMB), jnp.float32),
            jax.ShapeDtypeStruct((2 * NTOT, HEMB), jnp.float32),
        ),
        mesh=mesh,
        compiler_params=pltpu.CompilerParams(use_tc_tiling_on_sc=False),
        scratch_types=[
            pltpu.VMEM_SHARED((NTOT, HEMB), jnp.float32),
            pltpu.VMEM((3, B), jnp.int32),
            pltpu.VMEM((3, B), jnp.int32),
            pltpu.VMEM((B, HEMB), jnp.float32),
            pltpu.VMEM((B, HEMB), jnp.float32),
            pltpu.VMEM((ZCH, HEMB), jnp.float32),
            pltpu.VMEM((ZCH, HEMB), jnp.float32),
            pltpu.SemaphoreType.DMA,
            pltpu.SemaphoreType.DMA,
        ],
        name=f"lightgcn_layer_final{int(final)}",
    )


def kernel(edge_index, edge_weight, user_emb, item_emb):
    src = edge_index[1].astype(jnp.int32)
    dst = edge_index[0].astype(jnp.int32)
    src_p = src + PADG * (src >= N_U).astype(jnp.int32)
    dst_p = dst + PADG * (dst >= N_U).astype(jnp.int32)
    wbits = lax.bitcast_convert_type(edge_weight.astype(jnp.float32), jnp.int32)

    def pack(src_c):
        p = jnp.stack([src_c, dst_p, wbits])
        p = jnp.pad(p, ((0, 0), (0, E_PAD - E)))
        return p.reshape(3, NBALL, B).transpose(1, 0, 2)

    packed = jnp.concatenate([pack(src_p), pack(src_p + NTOT)], axis=0)

    zpad = jnp.zeros((PADG, EMB), jnp.float32)
    emb0 = jnp.concatenate([user_emb, zpad, item_emb, zpad], axis=0)
    split0 = jnp.concatenate([emb0[:, :HEMB], emb0[:, HEMB:]], axis=0)

    l_mid = _make_layer(False)
    l_last = _make_layer(True)
    emb1, sum1 = l_mid(packed, split0, split0)
    emb2, sum2 = l_mid(packed, emb1, sum1)
    _, sum3 = l_last(packed, emb2, sum2)

    out = jnp.concatenate([sum3[:NTOT], sum3[NTOT:]], axis=1)
    return (out[:N_U], out[HALF:HALF + N_I])


# speedup vs baseline: 7.0583x; 1.0100x over previous
import functools

import jax
import jax.numpy as jnp
from jax import lax
from jax.experimental import pallas as pl
from jax.experimental.pallas import tpu as pltpu
from jax.experimental.pallas import tpu_sc as plsc

N_U = 25000
N_I = 25000
EMB = 64
HEMB = 32
E = 800000

HALF = 25088
NTOT = 2 * HALF
PADG = HALF - N_U
B = 128
NBT = 392
NBALL = 16 * NBT
E_PAD = NBALL * B
STRIPE = NTOT // 16
ZCH = 196
NZ = STRIPE // ZCH


def _scale_rows(rows, pk):
    def group(g, carry):
        wv = lax.bitcast_convert_type(pk[2, pl.ds(g * 16, 16)], jnp.float32)
        for j in range(16):
            ws = wv.at[jnp.full((16,), j, jnp.int32)].get(
                mode="promise_in_bounds")
            e = g * 16 + j
            for q in range(HEMB // 16):
                sl = pl.ds(q * 16, 16)
                rows[e, sl] = rows[e, sl] * ws
        return carry

    lax.fori_loop(0, B // 16, group, 0)


def _one_layer(final, packed_hbm, tin_hbm, sin_hbm, tout_hbm, sout_hbm,
               acc, pk0, pk1, rows0, rows1, zbuf, sbuf, sem0, sem1):
    c = lax.axis_index("c")
    s = lax.axis_index("s")

    z = jnp.zeros((16,), jnp.float32)

    def zrow(r, carry):
        for q in range(HEMB // 16):
            zbuf[r, pl.ds(q * 16, 16)] = z
        return carry

    lax.fori_loop(0, ZCH, zrow, 0)
    for k in range(NZ):
        pltpu.sync_copy(zbuf, acc.at[pl.ds(s * STRIPE + k * ZCH, ZCH)])
    plsc.subcore_barrier()

    base = c * NBALL + s * NBT

    pltpu.sync_copy(packed_hbm.at[base], pk0)
    g0 = pltpu.async_copy(tin_hbm.at[pk0.at[0]], rows0, sem0)

    def pair(i, carry):
        b = 2 * i
        pltpu.sync_copy(packed_hbm.at[base + b + 1], pk1)
        pltpu.async_copy(tin_hbm.at[pk1.at[0]], rows1, sem1)
        pltpu.make_async_copy(tin_hbm.at[pk0.at[0]], rows0, sem0).wait()
        _scale_rows(rows0, pk0)
        pltpu.sync_copy(rows0, acc.at[pk0.at[1]], add=True)

        @pl.when(i < NBT // 2 - 1)
        def _():
            pltpu.sync_copy(packed_hbm.at[base + b + 2], pk0)
            pltpu.async_copy(tin_hbm.at[pk0.at[0]], rows0, sem0)

        pltpu.make_async_copy(tin_hbm.at[pk1.at[0]], rows1, sem1).wait()
        _scale_rows(rows1, pk1)
        pltpu.sync_copy(rows1, acc.at[pk1.at[1]], add=True)
        return carry

    lax.fori_loop(0, NBT // 2, pair, 0)
    plsc.subcore_barrier()

    row0 = c * NTOT + s * STRIPE
    if not final:
        pltpu.sync_copy(acc.at[pl.ds(s * STRIPE, STRIPE)],
                        tout_hbm.at[pl.ds(row0, STRIPE)])
    for k in range(NZ):
        pltpu.sync_copy(acc.at[pl.ds(s * STRIPE + k * ZCH, ZCH)], zbuf)
        pltpu.sync_copy(sin_hbm.at[pl.ds(row0 + k * ZCH, ZCH)], sbuf)

        def srow(r, carry):
            for q in range(HEMB // 16):
                sl = pl.ds(q * 16, 16)
                v = sbuf[r, sl] + zbuf[r, sl]
                if final:
                    v = v * 0.25
                sbuf[r, sl] = v
            return carry

        lax.fori_loop(0, ZCH, srow, 0)
        pltpu.sync_copy(sbuf, sout_hbm.at[pl.ds(row0 + k * ZCH, ZCH)])
    plsc.subcore_barrier()


def _fused_body(packed_hbm, t0_hbm, tmpa_hbm, tmpb_hbm, sum_hbm,
                acc, pk0, pk1, rows0, rows1, zbuf, sbuf, sem0, sem1):
    bufs = (acc, pk0, pk1, rows0, rows1, zbuf, sbuf, sem0, sem1)
    _one_layer(False, packed_hbm, t0_hbm, t0_hbm, tmpa_hbm, sum_hbm, *bufs)
    _one_layer(False, packed_hbm, tmpa_hbm, sum_hbm, tmpb_hbm, sum_hbm, *bufs)
    _one_layer(True, packed_hbm, tmpb_hbm, sum_hbm, None, sum_hbm, *bufs)


def _make_fused():
    mesh = plsc.VectorSubcoreMesh(core_axis_name="c", subcore_axis_name="s")
    return pl.kernel(
        _fused_body,
        out_type=(
            jax.ShapeDtypeStruct((2 * NTOT, HEMB), jnp.float32),
            jax.ShapeDtypeStruct((2 * NTOT, HEMB), jnp.float32),
            jax.ShapeDtypeStruct((2 * NTOT, HEMB), jnp.float32),
        ),
        mesh=mesh,
        compiler_params=pltpu.CompilerParams(use_tc_tiling_on_sc=False),
        scratch_types=[
            pltpu.VMEM_SHARED((NTOT, HEMB), jnp.float32),
            pltpu.VMEM((3, B), jnp.int32),
            pltpu.VMEM((3, B), jnp.int32),
            pltpu.VMEM((B, HEMB), jnp.float32),
            pltpu.VMEM((B, HEMB), jnp.float32),
            pltpu.VMEM((ZCH, HEMB), jnp.float32),
            pltpu.VMEM((ZCH, HEMB), jnp.float32),
            pltpu.SemaphoreType.DMA,
            pltpu.SemaphoreType.DMA,
        ],
        name="lightgcn_fused3",
    )


def kernel(edge_index, edge_weight, user_emb, item_emb):
    src = edge_index[1].astype(jnp.int32)
    dst = edge_index[0].astype(jnp.int32)
    src_p = src + PADG * (src >= N_U).astype(jnp.int32)
    dst_p = dst + PADG * (dst >= N_U).astype(jnp.int32)
    wbits = lax.bitcast_convert_type(edge_weight.astype(jnp.float32), jnp.int32)

    def pack(src_c):
        p = jnp.stack([src_c, dst_p, wbits])
        p = jnp.pad(p, ((0, 0), (0, E_PAD - E)))
        return p.reshape(3, NBALL, B).transpose(1, 0, 2)

    packed = jnp.concatenate([pack(src_p), pack(src_p + NTOT)], axis=0)

    zpad = jnp.zeros((PADG, EMB), jnp.float32)
    emb0 = jnp.concatenate([user_emb, zpad, item_emb, zpad], axis=0)
    split0 = jnp.concatenate([emb0[:, :HEMB], emb0[:, HEMB:]], axis=0)

    _, _, sum3 = _make_fused()(packed, split0)

    out = jnp.concatenate([sum3[:NTOT], sum3[NTOT:]], axis=1)
    return (out[:N_U], out[HALF:HALF + N_I])
